# Initial kernel scaffold; baseline (speedup 1.0000x reference)
import jax
import jax.numpy as jnp
from jax.experimental import pallas as pl


def kernel(x, edge_index, W0, a_src0, a_dst0, b0, bn_g0, bn_b0, W1, a_src1, a_dst1, b1, bn_g1, bn_b1, W2, a_src2, a_dst2, b2):
    raise NotImplementedError("write your pallas kernel here")


# speedup vs baseline: 6.4493x; 6.4493x over previous
import functools

import jax
import jax.numpy as jnp
from jax import lax
from jax.experimental import pallas as pl
from jax.experimental.pallas import tpu as pltpu
from jax.experimental.pallas import tpu_sc as plsc

N_NODES = 10000
NP = 10240
NWT = 32
ROWS = NP // NWT
E_RAW = 160000
E_TOT = E_RAW + N_NODES
EP = 174080
EBLK = 128
BN_EPS = 1e-5


def _sc_mesh():
    return plsc.VectorSubcoreMesh(core_axis_name="c", subcore_axis_name="s")


def _full16(v):
    return jnp.full((16,), v, jnp.int32)


def _prep_body(*refs, nc_in, nc_out, heads, relu):
    h_refs = refs[:nc_in]
    scale_ref, shift_ref, w_ref, as_ref, ad_ref = refs[nc_in:nc_in + 5]
    out_refs = refs[nc_in + 5:nc_in + 5 + nc_out]
    ts_ref, td_ref = refs[nc_in + 5 + nc_out:]
    z = jnp.concatenate([r[...] for r in h_refs], axis=-1)
    z = z * scale_ref[...] + shift_ref[...]
    if relu:
        z = jnp.maximum(z, 0.0)
    xw = jnp.dot(z, w_ref[...], preferred_element_type=jnp.float32)
    cw = xw.shape[1] // nc_out
    for c in range(nc_out):
        out_refs[c][...] = xw[:, c * cw:(c + 1) * cw]
    ts_ref[...] = jnp.dot(xw, as_ref[...], preferred_element_type=jnp.float32)
    td_ref[...] = jnp.dot(xw, ad_ref[...], preferred_element_type=jnp.float32)


def _tc_prep(h_chunks, scale, shift, W, As, Ad, relu):
    nc_in = len(h_chunks)
    k_in = sum(h.shape[1] for h in h_chunks)
    m_out = W.shape[1]
    heads = As.shape[1]
    nc_out = max(1, m_out // 128)
    cw = m_out // nc_out
    bn = 1024
    grid = (NP // bn,)
    in_specs = (
        [pl.BlockSpec((bn, h.shape[1]), lambda i: (i, 0)) for h in h_chunks]
        + [pl.BlockSpec((1, k_in), lambda i: (0, 0)),
           pl.BlockSpec((1, k_in), lambda i: (0, 0)),
           pl.BlockSpec((k_in, m_out), lambda i: (0, 0)),
           pl.BlockSpec((m_out, heads), lambda i: (0, 0)),
           pl.BlockSpec((m_out, heads), lambda i: (0, 0))]
    )
    out_specs = (
        [pl.BlockSpec((bn, cw), lambda i: (i, 0)) for _ in range(nc_out)]
        + [pl.BlockSpec((bn, heads), lambda i: (i, 0)),
           pl.BlockSpec((bn, heads), lambda i: (i, 0))]
    )
    out_shape = (
        [jax.ShapeDtypeStruct((NP, cw), jnp.float32) for _ in range(nc_out)]
        + [jax.ShapeDtypeStruct((NP, heads), jnp.float32),
           jax.ShapeDtypeStruct((NP, heads), jnp.float32)]
    )
    fn = pl.pallas_call(
        functools.partial(_prep_body, nc_in=nc_in, nc_out=nc_out,
                          heads=heads, relu=relu),
        grid=grid, in_specs=in_specs, out_specs=out_specs, out_shape=out_shape,
    )
    outs = fn(*h_chunks, scale.reshape(1, k_in), shift.reshape(1, k_in),
              W, As, Ad)
    return tuple(outs[:nc_out]), outs[nc_out], outs[nc_out + 1]


def _stats_body(*refs, nc):
    h_refs, out_ref = refs[:nc], refs[nc]
    i = pl.program_id(0)
    bs = h_refs[0].shape[0]
    rid = i * bs + lax.broadcasted_iota(jnp.int32, (bs, 1), 0)
    mask = rid < N_NODES

    @pl.when(i == 0)
    def _():
        out_ref[...] = jnp.zeros_like(out_ref)

    cw = h_refs[0].shape[1]
    for c in range(nc):
        z = jnp.where(mask, h_refs[c][...], 0.0)
        out_ref[0, c * cw:(c + 1) * cw] += jnp.sum(z, axis=0)
        out_ref[1, c * cw:(c + 1) * cw] += jnp.sum(z * z, axis=0)


def _tc_stats(h_chunks):
    nc = len(h_chunks)
    cw = h_chunks[0].shape[1]
    m = nc * cw
    bs = 1024
    fn = pl.pallas_call(
        functools.partial(_stats_body, nc=nc),
        grid=(NP // bs,),
        in_specs=[pl.BlockSpec((bs, cw), lambda i: (i, 0)) for _ in range(nc)],
        out_specs=pl.BlockSpec((8, m), lambda i: (0, 0)),
        out_shape=jax.ShapeDtypeStruct((8, m), jnp.float32),
    )
    return fn(*h_chunks)


def _make_sc_w(heads):
    mesh = _sc_mesh()

    @functools.partial(
        pl.kernel,
        out_type=jax.ShapeDtypeStruct((heads * EP,), jnp.float32),
        mesh=mesh,
        compiler_params=pltpu.CompilerParams(needs_layout_passes=False),
        scratch_types=[
            pltpu.VMEM((NP * heads,), jnp.float32),
            pltpu.VMEM((NP * heads,), jnp.float32),
            pltpu.VMEM((EBLK,), jnp.int32),
            pltpu.VMEM((EBLK,), jnp.int32),
            pltpu.VMEM((heads, EBLK), jnp.float32),
            pltpu.VMEM((128,), jnp.int32),
        ],
    )
    def k(ssrc_hbm, sdst_hbm, ts_hbm, td_hbm, bounds_hbm, w_hbm,
          ts_v, td_v, src_v, dst_v, w_v, bnd_v):
        wid = lax.axis_index("s") * 2 + lax.axis_index("c")
        pltpu.sync_copy(ts_hbm, ts_v)
        pltpu.sync_copy(td_hbm, td_v)
        pltpu.sync_copy(bounds_hbm, bnd_v)
        e0 = plsc.load_gather(bnd_v, [_full16(wid)])[0]
        e_real = plsc.load_gather(bnd_v, [_full16(64 + wid)])[0]
        e1 = plsc.load_gather(bnd_v, [_full16(wid + 1)])[0]
        e0 = pl.multiple_of(e0, 128)
        lane = lax.iota(jnp.int32, 16)

        def blk(b, _):
            a0 = pl.multiple_of(e0 + b * EBLK, 8)
            pltpu.sync_copy(ssrc_hbm.at[pl.ds(a0, EBLK)], src_v)
            pltpu.sync_copy(sdst_hbm.at[pl.ds(a0, EBLK)], dst_v)
            for g in range(EBLK // 16):
                s16 = src_v[pl.ds(g * 16, 16)]
                d16 = dst_v[pl.ds(g * 16, 16)]
                live = (a0 + g * 16 + lane) < e_real
                for h in range(heads):
                    a_s = plsc.load_gather(ts_v, [s16 * heads + h])
                    a_d = plsc.load_gather(td_v, [d16 * heads + h])
                    e = a_s + a_d
                    e = jnp.maximum(e, 0.2 * e)
                    w = jnp.where(live, jnp.exp(e), 0.0)
                    w_v[h, pl.ds(g * 16, 16)] = w
            for h in range(heads):
                pltpu.sync_copy(w_v.at[h], w_hbm.at[pl.ds(h * EP + a0, EBLK)])
            return 0

        lax.fori_loop(0, (e1 - e0) // EBLK, blk, 0)

    return k


def _make_sc_agg(heads, nc, cw):
    mesh = _sc_mesh()
    cph = nc // heads

    @functools.partial(
        pl.kernel,
        out_type=[jax.ShapeDtypeStruct((NP, cw), jnp.float32)
                  for _ in range(nc)],
        mesh=mesh,
        compiler_params=pltpu.CompilerParams(needs_layout_passes=False),
        scratch_types=[
            pltpu.VMEM((ROWS, cw), jnp.float32),
            pltpu.VMEM((ROWS, 16), jnp.float32),
            pltpu.VMEM((EBLK,), jnp.int32),
            pltpu.VMEM((EBLK + 16,), jnp.int32),
            pltpu.VMEM((EBLK + 16,), jnp.float32),
            pltpu.VMEM((EBLK, cw), jnp.float32),
            pltpu.VMEM((cw,), jnp.float32),
            pltpu.VMEM((128,), jnp.int32),
            pltpu.SemaphoreType.DMA,
        ],
    )
    def k(*refs):
        xw_hbm = refs[:nc]
        ssrc_hbm, sdst_hbm, w_hbm, bounds_hbm, bias_hbm = refs[nc:nc + 5]
        out_hbm = refs[nc + 5:nc + 5 + nc]
        (acc_v, wsum_v, src_v, dst_v, w_v, rows_v, bias_v, bnd_v,
         sem) = refs[nc + 5 + nc:]
        wid = lax.axis_index("s") * 2 + lax.axis_index("c")
        base = wid * ROWS
        pltpu.sync_copy(bounds_hbm, bnd_v)
        e0 = pl.multiple_of(plsc.load_gather(bnd_v, [_full16(wid)])[0], 128)
        e1 = plsc.load_gather(bnd_v, [_full16(wid + 1)])[0]
        nblk = (e1 - e0) // EBLK
        zero16 = jnp.zeros((16,), jnp.float32)
        lane0 = jnp.where(lax.iota(jnp.int32, 16) == 0, 1.0, 0.0)
        nv = cw // 16

        for c in range(nc):
            h = c // cph
            pltpu.sync_copy(bias_hbm.at[pl.ds(c * cw, cw)], bias_v)

            def zbody(r, _):
                for j in range(nv):
                    acc_v[r, pl.ds(j * 16, 16)] = zero16
                wsum_v[r, pl.ds(0, 16)] = zero16
                return 0

            lax.fori_loop(0, ROWS, zbody, 0)

            def blk(b, _):
                a0 = pl.multiple_of(e0 + b * EBLK, 8)
                pltpu.sync_copy(ssrc_hbm.at[pl.ds(a0, EBLK)], src_v)
                cp = pltpu.async_copy(xw_hbm[c].at[src_v], rows_v, sem)
                pltpu.sync_copy(sdst_hbm.at[pl.ds(a0, EBLK)],
                                dst_v.at[pl.ds(0, EBLK)])
                pltpu.sync_copy(w_hbm.at[pl.ds(h * EP + a0, EBLK)],
                                w_v.at[pl.ds(0, EBLK)])
                cp.wait()

                def egrp(g, _):
                    eb = g * 8
                    w8 = w_v[pl.ds(eb, 16)]
                    d8 = jnp.clip(dst_v[pl.ds(eb, 16)] - base, 0, ROWS - 1)
                    for i in range(8):
                        wv = w8[i]
                        dl = d8[i]
                        r = eb + i
                        for j in range(nv):
                            plsc.addupdate(
                                acc_v.at[dl, pl.ds(j * 16, 16)],
                                rows_v[r, pl.ds(j * 16, 16)] * wv)
                        plsc.addupdate(wsum_v.at[dl, pl.ds(0, 16)],
                                       wv * lane0)
                    return 0

                lax.fori_loop(0, EBLK // 8, egrp, 0)
                return 0

            lax.fori_loop(0, nblk, blk, 0)

            def nrm(r, _):
                winv = 1.0 / jnp.maximum(wsum_v[r, pl.ds(0, 16)], 1e-30)
                inv = winv[0]
                for j in range(nv):
                    acc_v[r, pl.ds(j * 16, 16)] = (
                        acc_v[r, pl.ds(j * 16, 16)] * inv
                        + bias_v[pl.ds(j * 16, 16)])
                return 0

            lax.fori_loop(0, ROWS, nrm, 0)
            pltpu.sync_copy(acc_v, out_hbm[c].at[pl.ds(base, ROWS)])

    return k


def _amat(a):
    h, c = a.shape
    m = jnp.zeros((h * c, h), jnp.float32)
    rows = (jnp.arange(h)[:, None] * c + jnp.arange(c)[None, :]).reshape(-1)
    cols = jnp.repeat(jnp.arange(h), c)
    return m.at[rows, cols].set(a.reshape(-1))


def _bn_affine(stats, gamma, beta):
    mu = stats[0] / N_NODES
    var = stats[1] / N_NODES - mu * mu
    scale = gamma * lax.rsqrt(var + BN_EPS)
    return scale, beta - mu * scale


def kernel(x, edge_index, W0, a_src0, a_dst0, b0, bn_g0, bn_b0,
           W1, a_src1, a_dst1, b1, bn_g1, bn_b1,
           W2, a_src2, a_dst2, b2):
    loops = jnp.arange(N_NODES, dtype=edge_index.dtype)
    src = jnp.concatenate([edge_index[0], loops]).astype(jnp.int32)
    dst = jnp.concatenate([edge_index[1], loops]).astype(jnp.int32)
    bkt = dst // ROWS
    counts = jnp.bincount(bkt, length=NWT).astype(jnp.int32)
    cnt128 = (counts + 127) // 128 * 128
    off = jnp.concatenate([jnp.zeros((1,), jnp.int32),
                           jnp.cumsum(cnt128).astype(jnp.int32)])
    onehot = (bkt[:, None] == jnp.arange(NWT)[None, :]).astype(jnp.int32)
    rank = jnp.take_along_axis(jnp.cumsum(onehot, axis=0),
                               bkt[:, None], axis=1)[:, 0] - 1
    pos = off[bkt] + rank
    ssrc = jnp.zeros((EP,), jnp.int32).at[pos].set(src)
    sdst = jnp.zeros((EP,), jnp.int32).at[pos].set(dst)
    bounds = (jnp.zeros((128,), jnp.int32)
              .at[0:NWT + 1].set(off)
              .at[64:64 + NWT].set(off[:NWT] + counts))

    x_pad = jnp.pad(x, ((0, NP - N_NODES), (0, 0)))
    ones256 = jnp.ones((256,), jnp.float32)
    zeros256 = jnp.zeros((256,), jnp.float32)

    sc_w4 = _make_sc_w(4)
    sc_w1 = _make_sc_w(1)
    sc_agg4 = _make_sc_agg(4, 8, 128)
    sc_agg1 = _make_sc_agg(1, 1, 128)

    xw0, ts0, td0 = _tc_prep((x_pad,), ones256, zeros256, W0,
                             _amat(a_src0), _amat(a_dst0), relu=False)
    w0 = sc_w4(ssrc, sdst, ts0.reshape(-1), td0.reshape(-1), bounds)
    h0 = sc_agg4(*xw0, ssrc, sdst, w0, bounds, b0)
    st0 = _tc_stats(h0)
    scale0, shift0 = _bn_affine(st0, bn_g0, bn_b0)

    xw1, ts1, td1 = _tc_prep(h0, scale0, shift0, W1,
                             _amat(a_src1), _amat(a_dst1), relu=True)
    w1 = sc_w4(ssrc, sdst, ts1.reshape(-1), td1.reshape(-1), bounds)
    h1 = sc_agg4(*xw1, ssrc, sdst, w1, bounds, b1)
    st1 = _tc_stats(h1)
    scale1, shift1 = _bn_affine(st1, bn_g1, bn_b1)

    xw2, ts2, td2 = _tc_prep(h1, scale1, shift1, W2,
                             _amat(a_src2), _amat(a_dst2), relu=True)
    w2 = sc_w1(ssrc, sdst, ts2.reshape(-1), td2.reshape(-1), bounds)
    out = sc_agg1(xw2[0], ssrc, sdst, w2, bounds, b2)
    return out[0][:N_NODES]
